# Initial kernel scaffold; baseline (speedup 1.0000x reference)
import jax
import jax.numpy as jnp
from jax.experimental import pallas as pl


def kernel(x, edge_index, edge_weight, weight):
    raise NotImplementedError("write your pallas kernel here")


# speedup vs baseline: 2.7909x; 2.7909x over previous
import jax
import jax.numpy as jnp
from jax import lax
from jax.experimental import pallas as pl
from jax.experimental.pallas import tpu as pltpu
from jax.experimental.pallas import tpu_sc as plsc

N = 10000
E = 160000
D = 256
H = 128
NC = 2
NS = 16
L = 16
K = 128
CH = 79
EP = NS * CH * K
RPT = N // NS
WR = 125
WCH = RPT // WR


def _mm_body(x_ref, w_ref, o_ref):
    o_ref[...] = jnp.dot(x_ref[...], w_ref[...],
                         preferred_element_type=jnp.float32)


def _matmul(x, weight):
    bn = 1000
    return pl.pallas_call(
        _mm_body,
        grid=(N // bn,),
        in_specs=[
            pl.BlockSpec((bn, D), lambda i: (i, 0)),
            pl.BlockSpec((D, D), lambda i: (0, 0)),
        ],
        out_specs=pl.BlockSpec((bn, D), lambda i: (i, 0)),
        out_shape=jax.ShapeDtypeStruct((N, D), jnp.float32),
    )(x, weight)


def _sc_body(xw2_hbm, src_hbm, dst_hbm, ew_hbm, widx_hbm, out_hbm,
             src_v, dst_v, ew_v, widx_v, buf, acc):
    wbuf = buf.at[pl.ds(0, WR)]
    c = lax.axis_index("c")
    s = lax.axis_index("s")

    pltpu.sync_copy(src_hbm.at[c, s], src_v)
    pltpu.sync_copy(dst_hbm.at[s], dst_v)
    pltpu.sync_copy(widx_hbm.at[c, s], widx_v)

    zeros = jnp.zeros((L,), jnp.float32)

    def zrow(r, _):
        for v in range(H // L):
            wbuf[r, pl.ds(v * L, L)] = zeros
        return 0

    lax.fori_loop(0, WR, zrow, 0)
    base = s * RPT

    def zcopy(k, _):
        pltpu.sync_copy(wbuf, acc.at[pl.ds(base + k * WR, WR)])
        return 0

    lax.fori_loop(0, WCH, zcopy, 0)
    plsc.subcore_barrier()

    def chunk_body(j, _):
        pltpu.sync_copy(xw2_hbm.at[src_v.at[j]], buf)
        pltpu.sync_copy(ew_hbm.at[s, j], ew_v)

        def edge_body(e, _):
            w = ew_v[pl.ds(e * L, L)]
            for v in range(H // L):
                sl = pl.ds(v * L, L)
                buf[e, sl] = buf[e, sl] * w
            return 0

        lax.fori_loop(0, K, edge_body, 0)
        pltpu.sync_copy(buf, acc.at[dst_v.at[j]], add=True)
        return 0

    lax.fori_loop(0, CH, chunk_body, 0)
    plsc.subcore_barrier()

    def wo_body(k, _):
        pltpu.sync_copy(acc.at[pl.ds(base + k * WR, WR)], wbuf)

        def relu_row(r, _):
            for v in range(H // L):
                sl = pl.ds(v * L, L)
                wbuf[r, sl] = jnp.maximum(wbuf[r, sl], 0.0)
            return 0

        lax.fori_loop(0, WR, relu_row, 0)
        pltpu.sync_copy(wbuf, out_hbm.at[widx_v.at[k]])
        return 0

    lax.fori_loop(0, WCH, wo_body, 0)


def _sc_scatter(xw2, src2, dst3, ew3, widx):
    mesh = plsc.VectorSubcoreMesh(core_axis_name="c", subcore_axis_name="s",
                                  num_cores=NC, num_subcores=NS)
    return pl.kernel(
        _sc_body,
        out_type=jax.ShapeDtypeStruct((2 * N, H), jnp.float32),
        mesh=mesh,
        scratch_types=[
            pltpu.VMEM((CH, K), jnp.int32),
            pltpu.VMEM((CH, K), jnp.int32),
            pltpu.VMEM((K * L,), jnp.float32),
            pltpu.VMEM((5, WR), jnp.int32),
            pltpu.VMEM((K, H), jnp.float32),
            pltpu.VMEM_SHARED((N, H), jnp.float32),
        ],
    )(xw2, src2, dst3, ew3, widx)


def kernel(x, edge_index, edge_weight, weight):
    xw = _matmul(x, weight)
    xw2 = xw.reshape(2 * N, H)

    src = edge_index[0]
    dst = edge_index[1]
    pad = EP - E
    srcp = jnp.pad(src, (0, pad))
    dstp = jnp.pad(dst, (0, pad))
    ewp = jnp.pad(edge_weight, (0, pad))

    core = jnp.arange(NC, dtype=jnp.int32)
    src2 = (2 * srcp[None, :] + core[:, None]).reshape(NC, NS, CH, K)
    dst3 = dstp.reshape(NS, CH, K)
    ew3 = jnp.broadcast_to(ewp[:, None], (EP, L)).reshape(NS, CH, K * L)

    rows = jnp.arange(N, dtype=jnp.int32).reshape(NS, WCH, WR)
    widx = (2 * rows[None] + core[:, None, None, None])

    out_flat = _sc_scatter(xw2, src2, dst3, ew3, widx)
    return out_flat.reshape(N, D)
